# baseline (speedup 1.0000x reference)
import functools

import jax
import jax.numpy as jnp
from jax import lax
from jax.experimental import pallas as pl
from jax.experimental.pallas import tpu as pltpu
from jax.experimental.pallas import tpu_sc as plsc

K = 64
D = 64
BETA = 0.25
BLOCK = 8192

_NC = 2
_NS = 16
_NW = _NC * _NS
_CHUNK = 128


def _vq_block_kernel(x_ref, e_ref, oh_ref, idx_ref, loss_ref):
    i = pl.program_id(0)
    x = x_ref[...]
    e = e_ref[...]

    w2 = jnp.sum(e * e, axis=0, keepdims=True)
    x2 = jnp.sum(x * x, axis=1, keepdims=True)
    s = jax.lax.dot_general(x, e, (((1,), (1,)), ((), ())),
                            preferred_element_type=jnp.float32)
    scores = (x2 + w2) - 2.0 * s

    m = jnp.min(scores, axis=1, keepdims=True)
    eq = (scores == m).astype(jnp.float32)

    kk = jax.lax.broadcasted_iota(jnp.int32, (K, K), 0)
    ll = jax.lax.broadcasted_iota(jnp.int32, (K, K), 1)
    lower = (kk < ll).astype(jnp.float32)
    prior = jax.lax.dot_general(eq, lower, (((1,), (0,)), ((), ())),
                                preferred_element_type=jnp.float32)
    one_hot = eq * (prior == 0.0).astype(jnp.float32)

    e2 = jnp.sum(e * e, axis=1, keepdims=True)
    col = jax.lax.broadcasted_iota(jnp.int32, (K, 1), 0).astype(jnp.float32)
    rhs = jnp.concatenate([col, e2 - w2.reshape(K, 1)], axis=1)
    sel = jax.lax.dot_general(one_hot, rhs, (((1,), (0,)), ((), ())),
                              preferred_element_type=jnp.float32)
    idx_ref[...] = sel[:, 0:1].astype(jnp.int32)

    loss_col = m + sel[:, 1:2]
    ones_row = jnp.ones((1, x.shape[0]), jnp.float32)
    partial = jax.lax.dot_general(ones_row, loss_col, (((1,), (0,)), ((), ())),
                                  preferred_element_type=jnp.float32)

    oh_ref[...] = one_hot

    @pl.when(i == 0)
    def _():
        loss_ref[...] = jnp.zeros_like(loss_ref)

    loss_ref[...] += partial


def _sc_gather_kernel(e_hbm, idx_hbm, out_hbm, idx_v, rows_v, sem):
    nchunks = idx_hbm.shape[1]
    c = lax.axis_index("c")
    s = lax.axis_index("s")
    wid = s * _NC + c
    rows_per_w = nchunks * _CHUNK
    base = wid * rows_per_w
    pltpu.sync_copy(idx_hbm.at[wid], idx_v)

    def body(j, carry):
        pltpu.async_copy(e_hbm.at[idx_v.at[j]], rows_v, sem).wait()
        pltpu.sync_copy(rows_v, out_hbm.at[pl.ds(base + j * _CHUNK, _CHUNK)])
        return carry

    lax.fori_loop(0, nchunks, body, 0)


def kernel(latents, embedding_weight):
    shape = latents.shape
    flat = latents.reshape(-1, D)
    n = flat.shape[0]
    nb = n // BLOCK

    one_hot, idx, loss = pl.pallas_call(
        _vq_block_kernel,
        grid=(nb,),
        in_specs=[
            pl.BlockSpec((BLOCK, D), lambda i: (i, 0)),
            pl.BlockSpec((K, D), lambda i: (0, 0)),
        ],
        out_specs=[
            pl.BlockSpec((BLOCK, K), lambda i: (i, 0)),
            pl.BlockSpec((BLOCK, 1), lambda i: (i, 0)),
            pl.BlockSpec((1, 1), lambda i: (0, 0)),
        ],
        out_shape=[
            jax.ShapeDtypeStruct((n, K), jnp.float32),
            jax.ShapeDtypeStruct((n, 1), jnp.int32),
            jax.ShapeDtypeStruct((1, 1), jnp.float32),
        ],
        compiler_params=pltpu.CompilerParams(
            dimension_semantics=("arbitrary",),
        ),
    )(flat, embedding_weight)

    nchunks = n // (_NW * _CHUNK)
    idx_sc = idx.reshape(_NW, nchunks, _CHUNK)
    gather = functools.partial(
        pl.kernel,
        out_type=jax.ShapeDtypeStruct((n, D), jnp.float32),
        mesh=plsc.VectorSubcoreMesh(core_axis_name="c", subcore_axis_name="s",
                                    num_cores=_NC, num_subcores=_NS),
        scratch_types=[
            pltpu.VMEM((nchunks, _CHUNK), jnp.int32),
            pltpu.VMEM((_CHUNK, D), jnp.float32),
            pltpu.SemaphoreType.DMA,
        ],
        compiler_params=pltpu.CompilerParams(use_tc_tiling_on_sc=False),
    )(_sc_gather_kernel)
    quant = gather(embedding_weight, idx_sc)

    quantized = quant.reshape(shape)
    indices = idx.reshape(shape[0], shape[1], shape[2])[:, None, :, :]
    vq_loss = loss[0, 0] * ((1.0 + BETA) / (n * D))
    return (quantized, vq_loss, one_hot, indices)


# speedup vs baseline: 1.0014x; 1.0014x over previous
import functools

import jax
import jax.numpy as jnp
from jax import lax
from jax.experimental import pallas as pl
from jax.experimental.pallas import tpu as pltpu
from jax.experimental.pallas import tpu_sc as plsc

K = 64
D = 64
BETA = 0.25
BLOCK = 8192

_NC = 2
_NS = 16
_NW = _NC * _NS
_CHUNK = 128


def _vq_block_kernel(x_ref, e_ref, oh_ref, idx_ref, loss_ref):
    i = pl.program_id(0)
    x = x_ref[...]
    e = e_ref[...]

    w2 = jnp.sum(e * e, axis=0, keepdims=True)
    x2 = jnp.sum(x * x, axis=1, keepdims=True)
    s = jax.lax.dot_general(x, e, (((1,), (1,)), ((), ())),
                            preferred_element_type=jnp.float32)
    scores = (x2 + w2) - 2.0 * s

    m = jnp.min(scores, axis=1, keepdims=True)
    eq = (scores == m).astype(jnp.float32)

    kk = jax.lax.broadcasted_iota(jnp.int32, (K, K), 0)
    ll = jax.lax.broadcasted_iota(jnp.int32, (K, K), 1)
    lower = (kk < ll).astype(jnp.float32)
    prior = jax.lax.dot_general(eq, lower, (((1,), (0,)), ((), ())),
                                preferred_element_type=jnp.float32)
    one_hot = eq * (prior == 0.0).astype(jnp.float32)

    e2 = jnp.sum(e * e, axis=1, keepdims=True)
    col = jax.lax.broadcasted_iota(jnp.int32, (K, 1), 0).astype(jnp.float32)
    rhs = jnp.concatenate([col, e2 - w2.reshape(K, 1)], axis=1)
    sel = jax.lax.dot_general(one_hot, rhs, (((1,), (0,)), ((), ())),
                              preferred_element_type=jnp.float32)
    idx_ref[...] = sel[:, 0:1].astype(jnp.int32)

    loss_col = m + sel[:, 1:2]
    ones_row = jnp.ones((1, x.shape[0]), jnp.float32)
    partial = jax.lax.dot_general(ones_row, loss_col, (((1,), (0,)), ((), ())),
                                  preferred_element_type=jnp.float32)

    oh_ref[...] = one_hot

    @pl.when(i == 0)
    def _():
        loss_ref[...] = jnp.zeros_like(loss_ref)

    loss_ref[...] += partial


_CPG = 4
_GROUP = _CPG * _CHUNK


def _sc_gather_kernel(e_hbm, idx_hbm, out_hbm, idx_v, buf0, buf1, gsem, wsem):
    nchunks = idx_hbm.shape[1]
    ngroups = nchunks // _CPG
    c = lax.axis_index("c")
    s = lax.axis_index("s")
    wid = s * _NC + c
    base = wid * nchunks * _CHUNK
    pltpu.sync_copy(idx_hbm.at[wid], idx_v)
    bufs = [buf0, buf1]

    def issue_gathers(g):
        buf = bufs[g % 2]
        return [
            pltpu.async_copy(e_hbm.at[idx_v.at[g * _CPG + jj]],
                             buf.at[pl.ds(jj * _CHUNK, _CHUNK)], gsem)
            for jj in range(_CPG)
        ]

    gather_descs = issue_gathers(0)
    write_descs = [None, None]
    for g in range(ngroups):
        buf = bufs[g % 2]
        if g + 1 < ngroups:
            nxt = bufs[(g + 1) % 2]
            if write_descs[(g + 1) % 2] is not None:
                write_descs[(g + 1) % 2].wait()
                write_descs[(g + 1) % 2] = None
            next_descs = issue_gathers(g + 1)
        else:
            next_descs = None
        for dsc in gather_descs:
            dsc.wait()
        write_descs[g % 2] = pltpu.async_copy(
            buf, out_hbm.at[pl.ds(base + g * _GROUP, _GROUP)], wsem)
        gather_descs = next_descs
    for wd in write_descs:
        if wd is not None:
            wd.wait()


def kernel(latents, embedding_weight):
    shape = latents.shape
    flat = latents.reshape(-1, D)
    n = flat.shape[0]
    nb = n // BLOCK

    one_hot, idx, loss = pl.pallas_call(
        _vq_block_kernel,
        grid=(nb,),
        in_specs=[
            pl.BlockSpec((BLOCK, D), lambda i: (i, 0)),
            pl.BlockSpec((K, D), lambda i: (0, 0)),
        ],
        out_specs=[
            pl.BlockSpec((BLOCK, K), lambda i: (i, 0)),
            pl.BlockSpec((BLOCK, 1), lambda i: (i, 0)),
            pl.BlockSpec((1, 1), lambda i: (0, 0)),
        ],
        out_shape=[
            jax.ShapeDtypeStruct((n, K), jnp.float32),
            jax.ShapeDtypeStruct((n, 1), jnp.int32),
            jax.ShapeDtypeStruct((1, 1), jnp.float32),
        ],
        compiler_params=pltpu.CompilerParams(
            dimension_semantics=("arbitrary",),
        ),
    )(flat, embedding_weight)

    nchunks = n // (_NW * _CHUNK)
    idx_sc = idx.reshape(_NW, nchunks, _CHUNK)
    gather = functools.partial(
        pl.kernel,
        out_type=jax.ShapeDtypeStruct((n, D), jnp.float32),
        mesh=plsc.VectorSubcoreMesh(core_axis_name="c", subcore_axis_name="s",
                                    num_cores=_NC, num_subcores=_NS),
        scratch_types=[
            pltpu.VMEM((nchunks, _CHUNK), jnp.int32),
            pltpu.VMEM((_GROUP, D), jnp.float32),
            pltpu.VMEM((_GROUP, D), jnp.float32),
            pltpu.SemaphoreType.DMA,
            pltpu.SemaphoreType.DMA,
        ],
        compiler_params=pltpu.CompilerParams(use_tc_tiling_on_sc=False),
    )(_sc_gather_kernel)
    quant = gather(embedding_weight, idx_sc)

    quantized = quant.reshape(shape)
    indices = idx.reshape(shape[0], shape[1], shape[2])[:, None, :, :]
    vq_loss = loss[0, 0] * ((1.0 + BETA) / (n * D))
    return (quantized, vq_loss, one_hot, indices)


# speedup vs baseline: 1.6787x; 1.6764x over previous
import functools

import jax
import jax.numpy as jnp
from jax import lax
from jax.experimental import pallas as pl
from jax.experimental.pallas import tpu as pltpu
from jax.experimental.pallas import tpu_sc as plsc

K = 64
D = 64
BETA = 0.25
BLOCK = 8192

_NC = 2
_NS = 16
_NW = _NC * _NS
_CHUNK = 128


def _vq_block_kernel(x_ref, e_ref, oh_ref, idx_ref, loss_ref):
    i = pl.program_id(0)
    x = x_ref[...]
    e = e_ref[...]

    w2 = jnp.sum(e * e, axis=0, keepdims=True)
    x2 = jnp.sum(x * x, axis=1, keepdims=True)
    s = jax.lax.dot_general(x, e, (((1,), (1,)), ((), ())),
                            preferred_element_type=jnp.float32)
    scores = (x2 + w2) - 2.0 * s

    m = jnp.min(scores, axis=1, keepdims=True)
    eq = (scores == m).astype(jnp.float32)

    kk = jax.lax.broadcasted_iota(jnp.int32, (K, K), 0)
    ll = jax.lax.broadcasted_iota(jnp.int32, (K, K), 1)
    lower = (kk < ll).astype(jnp.float32)
    prior = jax.lax.dot_general(eq, lower, (((1,), (0,)), ((), ())),
                                preferred_element_type=jnp.float32)
    one_hot = eq * (prior == 0.0).astype(jnp.float32)

    e2 = jnp.sum(e * e, axis=1, keepdims=True)
    col = jax.lax.broadcasted_iota(jnp.int32, (K, 1), 0).astype(jnp.float32)
    rhs = jnp.concatenate([col, e2 - w2.reshape(K, 1)], axis=1)
    sel = jax.lax.dot_general(one_hot, rhs, (((1,), (0,)), ((), ())),
                              preferred_element_type=jnp.float32)
    idx_ref[...] = sel[:, 0:1].astype(jnp.int32)

    loss_col = m + sel[:, 1:2]
    ones_row = jnp.ones((1, x.shape[0]), jnp.float32)
    partial = jax.lax.dot_general(ones_row, loss_col, (((1,), (0,)), ((), ())),
                                  preferred_element_type=jnp.float32)

    oh_ref[...] = one_hot

    @pl.when(i == 0)
    def _():
        loss_ref[...] = jnp.zeros_like(loss_ref)

    loss_ref[...] += partial


_GROUP = 512


def _sc_gather_kernel(e_hbm, idx_hbm, out_hbm, e_v, idx_v, buf0, buf1, wsem):
    rows_w = idx_hbm.shape[0] // _NW
    ngroups = rows_w // _GROUP
    c = lax.axis_index("c")
    s = lax.axis_index("s")
    wid = s * _NC + c
    base = wid * rows_w
    pltpu.sync_copy(e_hbm, e_v)
    pltpu.sync_copy(idx_hbm.at[pl.ds(base, rows_w)], idx_v)
    lane = jax.lax.broadcasted_iota(jnp.int32, (16,), 0)
    bufs = [buf0, buf1]
    wd = [None, None]

    for g in range(ngroups):
        buf = bufs[g % 2]
        if wd[g % 2] is not None:
            wd[g % 2].wait()
        g0 = g * _GROUP

        def body(r, g0=g0, buf=buf):
            ridx = plsc.load_gather(idx_v, [jnp.full((16,), g0 + r, jnp.int32)])
            for cc in range(D // 16):
                vals = plsc.load_gather(e_v, [ridx, lane + (cc * 16)])
                buf[r, pl.ds(cc * 16, 16)] = vals

        plsc.parallel_loop(0, _GROUP, unroll=8)(body)
        wd[g % 2] = pltpu.async_copy(
            buf, out_hbm.at[pl.ds(base + g0, _GROUP)], wsem)
    for d in wd:
        if d is not None:
            d.wait()


def kernel(latents, embedding_weight):
    shape = latents.shape
    flat = latents.reshape(-1, D)
    n = flat.shape[0]
    nb = n // BLOCK

    one_hot, idx, loss = pl.pallas_call(
        _vq_block_kernel,
        grid=(nb,),
        in_specs=[
            pl.BlockSpec((BLOCK, D), lambda i: (i, 0)),
            pl.BlockSpec((K, D), lambda i: (0, 0)),
        ],
        out_specs=[
            pl.BlockSpec((BLOCK, K), lambda i: (i, 0)),
            pl.BlockSpec((BLOCK, 1), lambda i: (i, 0)),
            pl.BlockSpec((1, 1), lambda i: (0, 0)),
        ],
        out_shape=[
            jax.ShapeDtypeStruct((n, K), jnp.float32),
            jax.ShapeDtypeStruct((n, 1), jnp.int32),
            jax.ShapeDtypeStruct((1, 1), jnp.float32),
        ],
        compiler_params=pltpu.CompilerParams(
            dimension_semantics=("arbitrary",),
        ),
    )(flat, embedding_weight)

    idx_sc = idx.reshape(n)
    gather = functools.partial(
        pl.kernel,
        out_type=jax.ShapeDtypeStruct((n, D), jnp.float32),
        mesh=plsc.VectorSubcoreMesh(core_axis_name="c", subcore_axis_name="s",
                                    num_cores=_NC, num_subcores=_NS),
        scratch_types=[
            pltpu.VMEM((K, D), jnp.float32),
            pltpu.VMEM((n // _NW,), jnp.int32),
            pltpu.VMEM((_GROUP, D), jnp.float32),
            pltpu.VMEM((_GROUP, D), jnp.float32),
            pltpu.SemaphoreType.DMA,
        ],
        compiler_params=pltpu.CompilerParams(use_tc_tiling_on_sc=False,
                                             needs_layout_passes=False),
    )(_sc_gather_kernel)
    quant = gather(embedding_weight, idx_sc)

    quantized = quant.reshape(shape)
    indices = idx.reshape(shape[0], shape[1], shape[2])[:, None, :, :]
    vq_loss = loss[0, 0] * ((1.0 + BETA) / (n * D))
    return (quantized, vq_loss, one_hot, indices)
